# baseline (speedup 1.0000x reference)
import functools

import jax
import jax.numpy as jnp
from jax import lax
from jax.experimental import pallas as pl
from jax.experimental.pallas import tpu as pltpu
from jax.experimental.pallas import tpu_sc as plsc

_NC = 2
_NS = 16
_NW = _NC * _NS
_CH = 128
_NBUF = 6


def _sc_gather(inputs1, inputs2, neg_samples, target_embeds, context_embeds,
               context_bias):
  V, D = target_embeds.shape
  B = inputs1.shape[0]
  NEG = neg_samples.shape[0]
  b_per_w = B // _NW
  n_ch = b_per_w // _CH
  neg_per_w = NEG // _NW

  mesh = plsc.VectorSubcoreMesh(core_axis_name="c", subcore_axis_name="s")

  @functools.partial(
      pl.kernel,
      out_type=(
          jax.ShapeDtypeStruct((B, D), jnp.float32),
          jax.ShapeDtypeStruct((B, D), jnp.float32),
          jax.ShapeDtypeStruct((B,), jnp.float32),
          jax.ShapeDtypeStruct((NEG, D), jnp.float32),
          jax.ShapeDtypeStruct((NEG,), jnp.float32),
      ),
      mesh=mesh,
      scratch_types=[
          pltpu.VMEM((n_ch, _CH), jnp.int32),
          pltpu.VMEM((n_ch, _CH), jnp.int32),
          pltpu.VMEM((neg_per_w,), jnp.int32),
          pltpu.VMEM((_NBUF, _CH, D), jnp.float32),
          pltpu.VMEM((n_ch, _CH), jnp.float32),
          pltpu.VMEM((neg_per_w, D), jnp.float32),
          pltpu.VMEM((neg_per_w,), jnp.float32),
          pltpu.SemaphoreType.DMA,
          [pltpu.SemaphoreType.DMA] * _NBUF,
          [pltpu.SemaphoreType.DMA] * _NBUF,
          pltpu.SemaphoreType.DMA,
      ],
  )
  def gather_kernel(i1_hbm, i2_hbm, ineg_hbm, tgt_hbm, ctx_hbm, cbias_hbm,
                    o1_hbm, o2_hbm, o2b_hbm, negr_hbm, negb_hbm,
                    idx1_v, idx2_v, idxn_v, rows_v, bias2_v, rowsn_v, biasn_v,
                    sem_i, sems_g, sems_o, sem_n):
    wid = lax.axis_index("s") * _NC + lax.axis_index("c")
    base = wid * b_per_w
    offn = wid * neg_per_w

    idx_copies = []
    for c in range(n_ch):
      off = base + c * _CH
      idx_copies.append(
          pltpu.async_copy(i1_hbm.at[pl.ds(off, _CH)], idx1_v.at[c], sem_i))
      idx_copies.append(
          pltpu.async_copy(i2_hbm.at[pl.ds(off, _CH)], idx2_v.at[c], sem_i))
    idx_copies.append(
        pltpu.async_copy(ineg_hbm.at[pl.ds(offn, neg_per_w)], idxn_v, sem_i))
    for cp in idx_copies:
      cp.wait()

    small = [
        pltpu.async_copy(ctx_hbm.at[idxn_v], rowsn_v, sem_n),
        pltpu.async_copy(cbias_hbm.at[idxn_v], biasn_v, sem_n),
    ]
    for c in range(n_ch):
      small.append(
          pltpu.async_copy(cbias_hbm.at[idx2_v.at[c]], bias2_v.at[c], sem_n))

    n_total = 2 * n_ch

    def chunk_gather(c, slot):
      if c < n_ch:
        return pltpu.async_copy(tgt_hbm.at[idx1_v.at[c]], rows_v.at[slot],
                                sems_g[slot])
      cc = c - n_ch
      return pltpu.async_copy(ctx_hbm.at[idx2_v.at[cc]], rows_v.at[slot],
                              sems_g[slot])

    def chunk_writeback(c, slot):
      if c < n_ch:
        dst = o1_hbm.at[pl.ds(base + c * _CH, _CH)]
      else:
        dst = o2_hbm.at[pl.ds(base + (c - n_ch) * _CH, _CH)]
      return pltpu.async_copy(rows_v.at[slot], dst, sems_o[slot])

    gathers = [chunk_gather(c, c) for c in range(min(_NBUF, n_total))]
    outs = [None] * _NBUF
    for c in range(n_total):
      slot = c % _NBUF
      gathers[slot].wait()
      outs[slot] = chunk_writeback(c, slot)
      nxt = c + _NBUF
      if nxt < n_total:
        outs[slot].wait()
        gathers[slot] = chunk_gather(nxt, slot)
    for cp in outs:
      if cp is not None:
        cp.wait()

    for cp in small:
      cp.wait()
    pltpu.sync_copy(rowsn_v, negr_hbm.at[pl.ds(offn, neg_per_w)])
    pltpu.sync_copy(biasn_v, negb_hbm.at[pl.ds(offn, neg_per_w)])
    for c in range(n_ch):
      pltpu.sync_copy(bias2_v.at[c], o2b_hbm.at[pl.ds(base + c * _CH, _CH)])

  return gather_kernel(inputs1, inputs2, neg_samples, target_embeds,
                       context_embeds, context_bias)


_BT = 1024


_LOG2E = 1.4426950408889634
_LN2 = 0.6931471805599453


def _softplus2_sum(t, group=1):
  t = jnp.minimum(t, 30.0)
  u = jnp.exp2(t)
  g = t.shape[0] // group
  w = 1.0 + u[0:g]
  for j in range(1, group):
    w = w * (1.0 + u[j * g:(j + 1) * g])
  return jnp.sum(jnp.log(w))


def _tc_loss_body(o1_ref, o2_ref, o2b_ref, neg_ref, negb_ref, out_ref):
  i = pl.program_id(0)
  o1 = o1_ref[...]
  prod = (o1 * o2_ref[...]).reshape(_BT // 128, 128, 128)
  aff = jnp.sum(prod, axis=2) + o2b_ref[...]
  logits2 = lax.dot_general(o1.astype(jnp.bfloat16), neg_ref[...],
                            (((1,), (1,)), ((), ())),
                            preferred_element_type=jnp.float32)
  part = (_softplus2_sum(aff * -_LOG2E) +
          _softplus2_sum(logits2 + negb_ref[...], group=4)).reshape(1, 1)

  @pl.when(i == 0)
  def _():
    out_ref[...] = jnp.zeros_like(out_ref)

  out_ref[...] += part


def _tc_loss(o1, o2, o2b, negr, negb):
  B, D = o1.shape
  NEG = negr.shape[0]
  grid = B // _BT
  out = pl.pallas_call(
      _tc_loss_body,
      grid=(grid,),
      in_specs=[
          pl.BlockSpec((_BT, D), lambda i: (i, 0)),
          pl.BlockSpec((_BT, D), lambda i: (i, 0)),
          pl.BlockSpec((_BT // 128, 128), lambda i: (i, 0)),
          pl.BlockSpec((NEG, D), lambda i: (0, 0)),
          pl.BlockSpec((1, NEG), lambda i: (0, 0)),
      ],
      out_specs=pl.BlockSpec((1, 1), lambda i: (0, 0)),
      out_shape=jax.ShapeDtypeStruct((1, 1), jnp.float32),
  )(o1, o2, o2b.reshape(B // 128, 128),
    (negr * jnp.float32(_LOG2E)).astype(jnp.bfloat16),
    negb.reshape(1, NEG) * jnp.float32(_LOG2E))
  return out[0, 0]


def kernel(inputs1, inputs2, neg_samples, target_embeds, context_embeds,
           context_bias):
  o1, o2, o2b, negr, negb = _sc_gather(inputs1, inputs2, neg_samples,
                                       target_embeds, context_embeds,
                                       context_bias)
  total = _tc_loss(o1, o2, o2b, negr, negb)
  return total / jnp.float32(inputs1.shape[0])


# speedup vs baseline: 1.1947x; 1.1947x over previous
import functools

import jax
import jax.numpy as jnp
from jax import lax
from jax.experimental import pallas as pl
from jax.experimental.pallas import tpu as pltpu
from jax.experimental.pallas import tpu_sc as plsc

_NC = 2
_NS = 16
_NW = _NC * _NS
_CH = 128
_NBUF = 6


def _sc_gather(inputs1, inputs2, target_embeds, context_embeds, context_bias,
               neg_samples=None):
  V, D = target_embeds.shape
  B = inputs1.shape[0]
  with_neg = neg_samples is not None
  NEG = neg_samples.shape[0] if with_neg else 0
  b_per_w = B // _NW
  n_ch = b_per_w // _CH
  neg_per_w = max(NEG // _NW, 1)
  nbuf = min(_NBUF, 2 * n_ch)

  mesh = plsc.VectorSubcoreMesh(core_axis_name="c", subcore_axis_name="s")

  out_type = [
      jax.ShapeDtypeStruct((B, D), jnp.float32),
      jax.ShapeDtypeStruct((B, D), jnp.float32),
      jax.ShapeDtypeStruct((B,), jnp.float32),
  ]
  if with_neg:
    out_type += [
        jax.ShapeDtypeStruct((NEG, D), jnp.float32),
        jax.ShapeDtypeStruct((NEG,), jnp.float32),
    ]

  scratch = [
      pltpu.VMEM((n_ch, _CH), jnp.int32),
      pltpu.VMEM((n_ch, _CH), jnp.int32),
      pltpu.VMEM((nbuf, _CH, D), jnp.float32),
      pltpu.VMEM((n_ch, _CH), jnp.float32),
      pltpu.SemaphoreType.DMA,
      [pltpu.SemaphoreType.DMA] * nbuf,
      [pltpu.SemaphoreType.DMA] * nbuf,
      pltpu.SemaphoreType.DMA,
  ]
  if with_neg:
    scratch += [
        pltpu.VMEM((neg_per_w,), jnp.int32),
        pltpu.VMEM((neg_per_w, D), jnp.float32),
        pltpu.VMEM((neg_per_w,), jnp.float32),
    ]

  @functools.partial(pl.kernel, out_type=tuple(out_type), mesh=mesh,
                     scratch_types=scratch)
  def gather_kernel(*refs):
    if with_neg:
      (i1_hbm, i2_hbm, ineg_hbm, tgt_hbm, ctx_hbm, cbias_hbm,
       o1_hbm, o2_hbm, o2b_hbm, negr_hbm, negb_hbm,
       idx1_v, idx2_v, rows_v, bias2_v,
       sem_i, sems_g, sems_o, sem_n, idxn_v, rowsn_v, biasn_v) = refs
    else:
      (i1_hbm, i2_hbm, tgt_hbm, ctx_hbm, cbias_hbm,
       o1_hbm, o2_hbm, o2b_hbm,
       idx1_v, idx2_v, rows_v, bias2_v,
       sem_i, sems_g, sems_o, sem_n) = refs
    wid = lax.axis_index("s") * _NC + lax.axis_index("c")
    base = wid * b_per_w
    offn = wid * neg_per_w

    idx_copies = []
    for c in range(n_ch):
      off = base + c * _CH
      idx_copies.append(
          pltpu.async_copy(i1_hbm.at[pl.ds(off, _CH)], idx1_v.at[c], sem_i))
      idx_copies.append(
          pltpu.async_copy(i2_hbm.at[pl.ds(off, _CH)], idx2_v.at[c], sem_i))
    if with_neg:
      idx_copies.append(
          pltpu.async_copy(ineg_hbm.at[pl.ds(offn, neg_per_w)], idxn_v, sem_i))
    for cp in idx_copies:
      cp.wait()

    small = []
    if with_neg:
      small.append(pltpu.async_copy(ctx_hbm.at[idxn_v], rowsn_v, sem_n))
      small.append(pltpu.async_copy(cbias_hbm.at[idxn_v], biasn_v, sem_n))
    for c in range(n_ch):
      small.append(
          pltpu.async_copy(cbias_hbm.at[idx2_v.at[c]], bias2_v.at[c], sem_n))

    n_total = 2 * n_ch

    def chunk_gather(c, slot):
      if c < n_ch:
        return pltpu.async_copy(tgt_hbm.at[idx1_v.at[c]], rows_v.at[slot],
                                sems_g[slot])
      cc = c - n_ch
      return pltpu.async_copy(ctx_hbm.at[idx2_v.at[cc]], rows_v.at[slot],
                              sems_g[slot])

    def chunk_writeback(c, slot):
      if c < n_ch:
        dst = o1_hbm.at[pl.ds(base + c * _CH, _CH)]
      else:
        dst = o2_hbm.at[pl.ds(base + (c - n_ch) * _CH, _CH)]
      return pltpu.async_copy(rows_v.at[slot], dst, sems_o[slot])

    gathers = [chunk_gather(c, c) for c in range(min(nbuf, n_total))]
    outs = [None] * nbuf
    for c in range(n_total):
      slot = c % nbuf
      gathers[slot].wait()
      outs[slot] = chunk_writeback(c, slot)
      nxt = c + nbuf
      if nxt < n_total:
        outs[slot].wait()
        gathers[slot] = chunk_gather(nxt, slot)
    for cp in outs:
      if cp is not None:
        cp.wait()

    for cp in small:
      cp.wait()
    if with_neg:
      pltpu.sync_copy(rowsn_v, negr_hbm.at[pl.ds(offn, neg_per_w)])
      pltpu.sync_copy(biasn_v, negb_hbm.at[pl.ds(offn, neg_per_w)])
    for c in range(n_ch):
      pltpu.sync_copy(bias2_v.at[c], o2b_hbm.at[pl.ds(base + c * _CH, _CH)])

  if with_neg:
    return gather_kernel(inputs1, inputs2, neg_samples, target_embeds,
                         context_embeds, context_bias)
  return gather_kernel(inputs1, inputs2, target_embeds, context_embeds,
                       context_bias)


_BT = 1024


_LOG2E = 1.4426950408889634
_LN2 = 0.6931471805599453


def _softplus2_sum(t, group=1):
  t = jnp.minimum(t, 30.0)
  u = jnp.exp2(t)
  g = t.shape[0] // group
  w = 1.0 + u[0:g]
  for j in range(1, group):
    w = w * (1.0 + u[j * g:(j + 1) * g])
  return jnp.sum(jnp.log(w))


def _tc_loss_body(o1_ref, o2_ref, o2b_ref, neg_ref, negb_ref, out_ref):
  i = pl.program_id(0)
  o1 = o1_ref[...]
  prod = (o1 * o2_ref[...]).reshape(_BT // 128, 128, 128)
  aff = jnp.sum(prod, axis=2) + o2b_ref[...]
  logits2 = lax.dot_general(o1.astype(jnp.bfloat16), neg_ref[...],
                            (((1,), (1,)), ((), ())),
                            preferred_element_type=jnp.float32)
  part = (_softplus2_sum(aff * -_LOG2E) +
          _softplus2_sum(logits2 + negb_ref[...], group=4)).reshape(1, 1)

  @pl.when(i == 0)
  def _():
    out_ref[...] = jnp.zeros_like(out_ref)

  out_ref[...] += part


def _tc_loss(o1, o2, o2b, negr2, negb2):
  B, D = o1.shape
  NEG = negr2.shape[1]
  grid = B // _BT
  out = pl.pallas_call(
      _tc_loss_body,
      grid=(grid,),
      in_specs=[
          pl.BlockSpec((_BT, D), lambda i: (i, 0)),
          pl.BlockSpec((_BT, D), lambda i: (i, 0)),
          pl.BlockSpec((_BT // 128, 128), lambda i: (i, 0)),
          pl.BlockSpec((NEG, D), lambda i: (0, 0)),
          pl.BlockSpec((1, NEG), lambda i: (0, 0)),
      ],
      out_specs=pl.BlockSpec((1, 1), lambda i: (0, 0)),
      out_shape=jax.ShapeDtypeStruct((1, 1), jnp.float32),
  )(o1, o2, o2b.reshape(B // 128, 128), negr2, negb2)
  return out


def kernel(inputs1, inputs2, neg_samples, target_embeds, context_embeds,
           context_bias):
  B = inputs1.shape[0]
  NEG = neg_samples.shape[0]
  h = B // 2
  o1a, o2a, o2ba, negr, negb = _sc_gather(
      inputs1[:h], inputs2[:h], target_embeds, context_embeds, context_bias,
      neg_samples=neg_samples)
  o1b, o2b_, o2bb = _sc_gather(
      inputs1[h:], inputs2[h:], target_embeds, context_embeds, context_bias)
  negr2 = (negr * jnp.float32(_LOG2E)).astype(jnp.bfloat16)
  negb2 = negb.reshape(1, NEG) * jnp.float32(_LOG2E)
  pa = _tc_loss(o1a, o2a, o2ba, negr2, negb2)
  pb = _tc_loss(o1b, o2b_, o2bb, negr2, negb2)
  return (pa[0, 0] + pb[0, 0]) / jnp.float32(B)
